# Initial kernel scaffold; baseline (speedup 1.0000x reference)
import jax
import jax.numpy as jnp
from jax.experimental import pallas as pl


def kernel(feat, graphs, W1, b1, W2, b2, Wih1, Whh1, bih1, bhh1, Wih2, Whh2, bih2, bhh2):
    raise NotImplementedError("write your pallas kernel here")


# speedup vs baseline: 13.2904x; 13.2904x over previous
import functools

import jax
import jax.numpy as jnp
from jax import lax
from jax.experimental import pallas as pl
from jax.experimental.pallas import tpu as pltpu
from jax.experimental.pallas import tpu_sc as plsc

_N = 10000
_E = 320000
_T = 3
_DIN = 128
_DH = 256
_HALF = 128

_NC = 2
_NS = 16
_LANES = 16

_TSTRIDE = 10240
_CHD = 2000
_EPT_DEG = _E // (_NC * _NS)

_CH = 80
_KB = 4
_SUP = _CH * _KB
_NSUP = _E // _SUP
_RPT = 640


def _sc_mesh():
  return plsc.VectorSubcoreMesh(
      core_axis_name="c", subcore_axis_name="s",
      num_cores=_NC, num_subcores=_NS)


def _sc_degree(rows2d):
  @functools.partial(
      pl.kernel,
      out_type=jax.ShapeDtypeStruct((_NC * _NS, _T * _TSTRIDE), jnp.float32),
      mesh=_sc_mesh(),
      scratch_types=[
          pltpu.VMEM((_CHD,), jnp.int32),
          pltpu.VMEM((_T * _TSTRIDE,), jnp.float32),
      ],
      compiler_params=pltpu.CompilerParams(needs_layout_passes=False),
  )
  def k(rows_hbm, out_hbm, idx_v, deg_v):
    cid = lax.axis_index("c")
    sid = lax.axis_index("s")
    wid = sid * _NC + cid
    zeros16 = jnp.zeros((_LANES,), jnp.float32)
    ones16 = jnp.ones((_LANES,), jnp.float32)

    @pl.loop(0, _T * _TSTRIDE // _LANES)
    def _(i):
      deg_v[pl.ds(i * _LANES, _LANES)] = zeros16

    for t in range(_T):
      base_blk = t * (_E // _CHD) + wid * (_EPT_DEG // _CHD)
      off16 = jnp.full((_LANES,), t * _TSTRIDE, jnp.int32)
      for b in range(_EPT_DEG // _CHD):
        pltpu.sync_copy(rows_hbm.at[base_blk + b], idx_v)

        @pl.loop(0, _CHD // _LANES)
        def _(j):
          v = idx_v[pl.ds(j * _LANES, _LANES)] + off16
          plsc.addupdate_scatter(deg_v, [v], ones16)

    pltpu.sync_copy(deg_v, out_hbm.at[wid])

  return k(rows2d)


def _sc_scatter(edges3d, y0, y1):
  @functools.partial(
      pl.kernel,
      out_type=[
          jax.ShapeDtypeStruct((_N, _HALF), jnp.float32),
          jax.ShapeDtypeStruct((_N, _HALF), jnp.float32),
      ],
      mesh=_sc_mesh(),
      scratch_types=[
          pltpu.VMEM_SHARED((_N, _HALF), jnp.float32),
          pltpu.VMEM((2 * _KB, _CH), jnp.int32),
          [pltpu.VMEM((_CH, _HALF), jnp.float32) for _ in range(_KB)],
          pltpu.SemaphoreType.DMA,
          pltpu.SemaphoreType.DMA,
      ],
      compiler_params=pltpu.CompilerParams(needs_layout_passes=False),
  )
  def k(edges_hbm, y0_hbm, y1_hbm, agg0_hbm, agg1_hbm,
        acc, idx_rc, gat, gsem, ssem):
    cid = lax.axis_index("c")
    sid = lax.axis_index("s")
    last = _NS - 1
    tail = _N - last * _RPT

    def init_acc(y_hbm):
      @pl.when(sid < last)
      def _():
        pltpu.sync_copy(y_hbm.at[pl.ds(sid * _RPT, _RPT)],
                        acc.at[pl.ds(sid * _RPT, _RPT)])
      @pl.when(sid == last)
      def _():
        pltpu.sync_copy(y_hbm.at[pl.ds(last * _RPT, tail)],
                        acc.at[pl.ds(last * _RPT, tail)])

    def edge_loop(y_hbm):
      rem = _NSUP - (_NSUP // _NS) * _NS
      ntr = (_NSUP // _NS) + jnp.where(sid < rem, 1, 0)

      @pl.loop(0, ntr)
      def _(si):
        blk = si * _NS + sid
        pltpu.sync_copy(edges_hbm.at[blk], idx_rc)
        gd = [pltpu.async_copy(y_hbm.at[idx_rc.at[j]], gat[j], gsem)
              for j in range(_KB)]
        for d in gd:
          d.wait()
        sd = [pltpu.async_copy(gat[j], acc.at[idx_rc.at[_KB + j]], ssem,
                               add=True)
              for j in range(_KB)]
        for d in sd:
          d.wait()

    def copy_out(agg_hbm):
      @pl.when(sid < last)
      def _():
        pltpu.sync_copy(acc.at[pl.ds(sid * _RPT, _RPT)],
                        agg_hbm.at[pl.ds(sid * _RPT, _RPT)])
      @pl.when(sid == last)
      def _():
        pltpu.sync_copy(acc.at[pl.ds(last * _RPT, tail)],
                        agg_hbm.at[pl.ds(last * _RPT, tail)])

    @pl.when(cid == 0)
    def _():
      init_acc(y0_hbm)
    @pl.when(cid == 1)
    def _():
      init_acc(y1_hbm)
    plsc.subcore_barrier()
    @pl.when(cid == 0)
    def _():
      edge_loop(y0_hbm)
    @pl.when(cid == 1)
    def _():
      edge_loop(y1_hbm)
    plsc.subcore_barrier()
    @pl.when(cid == 0)
    def _():
      copy_out(agg0_hbm)
    @pl.when(cid == 1)
    def _():
      copy_out(agg1_hbm)

  return k(edges3d, y0, y1)


_R = 1000


def _tc_params():
  return pltpu.CompilerParams(dimension_semantics=("parallel",))


def _tc_dinv(deg_part):
  blk = 1280
  grid = (_T * _TSTRIDE) // blk

  def body(p_ref, o_ref):
    s = jnp.sum(p_ref[...], axis=0, keepdims=True) + 1.0
    o_ref[...] = lax.rsqrt(s)

  return pl.pallas_call(
      body,
      grid=(grid,),
      in_specs=[pl.BlockSpec((_NC * _NS, blk), lambda i: (0, i))],
      out_specs=pl.BlockSpec((1, blk), lambda i: (0, i)),
      out_shape=jax.ShapeDtypeStruct((1, _T * _TSTRIDE), jnp.float32),
      compiler_params=_tc_params(),
  )(deg_part)


def _tc_x1(feat, W1):
  def body(f_ref, w_ref, o_ref):
    o_ref[...] = jnp.dot(f_ref[...], w_ref[...],
                         preferred_element_type=jnp.float32)

  return pl.pallas_call(
      body,
      grid=(_N // _R,),
      in_specs=[
          pl.BlockSpec((_R, _DIN), lambda i: (i, 0)),
          pl.BlockSpec((_DIN, _DH), lambda i: (0, 0)),
      ],
      out_specs=pl.BlockSpec((_R, _DH), lambda i: (i, 0)),
      out_shape=jax.ShapeDtypeStruct((_N, _DH), jnp.float32),
      compiler_params=_tc_params(),
  )(feat, W1)


def _tc_prep(X1, dv):
  def body(x_ref, d_ref, y0_ref, y1_ref):
    y = x_ref[...] * d_ref[...]
    y0_ref[...] = y[:, :_HALF]
    y1_ref[...] = y[:, _HALF:]

  return pl.pallas_call(
      body,
      grid=(_N // _R,),
      in_specs=[
          pl.BlockSpec((_R, _DH), lambda i: (i, 0)),
          pl.BlockSpec((_R, 1), lambda i: (i, 0)),
      ],
      out_specs=[
          pl.BlockSpec((_R, _HALF), lambda i: (i, 0)),
          pl.BlockSpec((_R, _HALF), lambda i: (i, 0)),
      ],
      out_shape=[
          jax.ShapeDtypeStruct((_N, _HALF), jnp.float32),
          jax.ShapeDtypeStruct((_N, _HALF), jnp.float32),
      ],
      compiler_params=_tc_params(),
  )(X1, dv)


def _gru_math(h, hid, WihT, WhhT, bih, bhh):
  gi = jnp.dot(h, WihT, preferred_element_type=jnp.float32) + bih
  gh = jnp.dot(hid, WhhT, preferred_element_type=jnp.float32) + bhh
  r = jax.nn.sigmoid(gi[:, :_DH] + gh[:, :_DH])
  z = jax.nn.sigmoid(gi[:, _DH:2 * _DH] + gh[:, _DH:2 * _DH])
  n = jnp.tanh(gi[:, 2 * _DH:] + r * gh[:, 2 * _DH:])
  return (1.0 - z) * n + z * hid


def _tc_fused1(first, agg0, agg1, dv, hid, WihT, WhhT, bih, bhh, b, W2):
  def body(*refs):
    if first:
      (a0_ref, a1_ref, d_ref, wi_ref, wh_ref, bi_ref, bh_ref, b_ref, w2_ref,
       g_ref, z0_ref, z1_ref) = refs
    else:
      (a0_ref, a1_ref, d_ref, hid_ref, wi_ref, wh_ref, bi_ref, bh_ref, b_ref,
       w2_ref, g_ref, z0_ref, z1_ref) = refs
    dvb = d_ref[...]
    h = jnp.concatenate([a0_ref[...], a1_ref[...]], axis=1) * dvb + b_ref[...]
    hidv = h if first else hid_ref[...]
    g = _gru_math(h, hidv, wi_ref[...], wh_ref[...], bi_ref[...], bh_ref[...])
    g_ref[...] = g
    y2 = jnp.dot(g, w2_ref[...], preferred_element_type=jnp.float32) * dvb
    z0_ref[...] = y2[:, :_HALF]
    z1_ref[...] = y2[:, _HALF:]

  half_spec = pl.BlockSpec((_R, _HALF), lambda i: (i, 0))
  in_specs = [half_spec, half_spec, pl.BlockSpec((_R, 1), lambda i: (i, 0))]
  ins = [agg0, agg1, dv]
  if not first:
    in_specs.append(pl.BlockSpec((_R, _DH), lambda i: (i, 0)))
    ins.append(hid)
  in_specs += [
      pl.BlockSpec((_DH, 3 * _DH), lambda i: (0, 0)),
      pl.BlockSpec((_DH, 3 * _DH), lambda i: (0, 0)),
      pl.BlockSpec((1, 3 * _DH), lambda i: (0, 0)),
      pl.BlockSpec((1, 3 * _DH), lambda i: (0, 0)),
      pl.BlockSpec((1, _DH), lambda i: (0, 0)),
      pl.BlockSpec((_DH, _DH), lambda i: (0, 0)),
  ]
  ins += [WihT, WhhT, bih, bhh, b, W2]
  return pl.pallas_call(
      body,
      grid=(_N // _R,),
      in_specs=in_specs,
      out_specs=[
          pl.BlockSpec((_R, _DH), lambda i: (i, 0)),
          half_spec,
          half_spec,
      ],
      out_shape=[
          jax.ShapeDtypeStruct((_N, _DH), jnp.float32),
          jax.ShapeDtypeStruct((_N, _HALF), jnp.float32),
          jax.ShapeDtypeStruct((_N, _HALF), jnp.float32),
      ],
      compiler_params=_tc_params(),
  )(*ins)


def _tc_fused2(first, last, agg0, agg1, dv, hid, WihT, WhhT, bih, bhh, b,
               X1, dvn):
  def body(*refs):
    refs = list(refs)
    a0_ref = refs.pop(0)
    a1_ref = refs.pop(0)
    d_ref = refs.pop(0)
    hid_ref = None if first else refs.pop(0)
    wi_ref = refs.pop(0)
    wh_ref = refs.pop(0)
    bi_ref = refs.pop(0)
    bh_ref = refs.pop(0)
    b_ref = refs.pop(0)
    if not last:
      x1_ref = refs.pop(0)
      dn_ref = refs.pop(0)
    g_ref = refs.pop(0)
    dvb = d_ref[...]
    h = jnp.concatenate([a0_ref[...], a1_ref[...]], axis=1) * dvb + b_ref[...]
    hidv = h if first else hid_ref[...]
    g = _gru_math(h, hidv, wi_ref[...], wh_ref[...], bi_ref[...], bh_ref[...])
    g_ref[...] = g
    if not last:
      y0_ref = refs.pop(0)
      y1_ref = refs.pop(0)
      yn = x1_ref[...] * dn_ref[...]
      y0_ref[...] = yn[:, :_HALF]
      y1_ref[...] = yn[:, _HALF:]

  half_spec = pl.BlockSpec((_R, _HALF), lambda i: (i, 0))
  dv_spec = pl.BlockSpec((_R, 1), lambda i: (i, 0))
  in_specs = [half_spec, half_spec, dv_spec]
  ins = [agg0, agg1, dv]
  if not first:
    in_specs.append(pl.BlockSpec((_R, _DH), lambda i: (i, 0)))
    ins.append(hid)
  in_specs += [
      pl.BlockSpec((_DH, 3 * _DH), lambda i: (0, 0)),
      pl.BlockSpec((_DH, 3 * _DH), lambda i: (0, 0)),
      pl.BlockSpec((1, 3 * _DH), lambda i: (0, 0)),
      pl.BlockSpec((1, 3 * _DH), lambda i: (0, 0)),
      pl.BlockSpec((1, _DH), lambda i: (0, 0)),
  ]
  ins += [WihT, WhhT, bih, bhh, b]
  if not last:
    in_specs += [pl.BlockSpec((_R, _DH), lambda i: (i, 0)), dv_spec]
    ins += [X1, dvn]
  out_specs = [pl.BlockSpec((_R, _DH), lambda i: (i, 0))]
  out_shape = [jax.ShapeDtypeStruct((_N, _DH), jnp.float32)]
  if not last:
    out_specs += [half_spec, half_spec]
    out_shape += [
        jax.ShapeDtypeStruct((_N, _HALF), jnp.float32),
        jax.ShapeDtypeStruct((_N, _HALF), jnp.float32),
    ]
  res = pl.pallas_call(
      body,
      grid=(_N // _R,),
      in_specs=in_specs,
      out_specs=out_specs,
      out_shape=out_shape,
      compiler_params=_tc_params(),
  )(*ins)
  return res[0] if last else res


def kernel(feat, graphs, W1, b1, W2, b2, Wih1, Whh1, bih1, bhh1,
           Wih2, Whh2, bih2, bhh2):
  rows_all = graphs[:, 0, :].reshape(_T * _E // _CHD, _CHD)
  deg_part = _sc_degree(rows_all)
  dinv = _tc_dinv(deg_part).reshape(_T, _TSTRIDE)

  X1 = _tc_x1(feat, W1)
  Wih1T, Whh1T = Wih1.T, Whh1.T
  Wih2T, Whh2T = Wih2.T, Whh2.T
  bih1r, bhh1r = bih1.reshape(1, -1), bhh1.reshape(1, -1)
  bih2r, bhh2r = bih2.reshape(1, -1), bhh2.reshape(1, -1)
  b1r, b2r = b1.reshape(1, -1), b2.reshape(1, -1)

  y0, y1h = _tc_prep(X1, dinv[0].reshape(_TSTRIDE, 1))
  g1 = None
  g2 = None
  outs = []
  for t in range(_T):
    r3 = graphs[t, 0].reshape(_NSUP, _KB, _CH)
    c3 = graphs[t, 1].reshape(_NSUP, _KB, _CH)
    edges3d = jnp.concatenate([r3, c3], axis=1)
    dv = dinv[t].reshape(_TSTRIDE, 1)

    agg0, agg1 = _sc_scatter(edges3d, y0, y1h)
    g1, z0, z1 = _tc_fused1(t == 0, agg0, agg1, dv, g1,
                            Wih1T, Whh1T, bih1r, bhh1r, b1r, W2)

    agg20, agg21 = _sc_scatter(edges3d, z0, z1)
    if t < _T - 1:
      dvn = dinv[t + 1].reshape(_TSTRIDE, 1)
      g2, y0, y1h = _tc_fused2(t == 0, False, agg20, agg21, dv, g2,
                               Wih2T, Whh2T, bih2r, bhh2r, b2r, X1, dvn)
    else:
      g2 = _tc_fused2(t == 0, True, agg20, agg21, dv, g2,
                      Wih2T, Whh2T, bih2r, bhh2r, b2r, None, None)
    outs.append(g2)
  return jnp.stack(outs)


# speedup vs baseline: 14.2083x; 1.0691x over previous
import functools

import jax
import jax.numpy as jnp
from jax import lax
from jax.experimental import pallas as pl
from jax.experimental.pallas import tpu as pltpu
from jax.experimental.pallas import tpu_sc as plsc

_N = 10000
_E = 320000
_T = 3
_DIN = 128
_DH = 256
_HALF = 128

_NC = 2
_NS = 16
_LANES = 16

_TSTRIDE = 10240
_CHD = 2000
_EPT_DEG = _E // (_NC * _NS)

_CH = 80
_KB = 2
_SUP = _CH * _KB
_NSUP = _E // _SUP
_NCPT = _NSUP // _NS
_RPT = 640


def _sc_mesh():
  return plsc.VectorSubcoreMesh(
      core_axis_name="c", subcore_axis_name="s",
      num_cores=_NC, num_subcores=_NS)


def _sc_degree(rows2d):
  @functools.partial(
      pl.kernel,
      out_type=jax.ShapeDtypeStruct((_NC * _NS, _T * _TSTRIDE), jnp.float32),
      mesh=_sc_mesh(),
      scratch_types=[
          pltpu.VMEM((_CHD,), jnp.int32),
          pltpu.VMEM((_T * _TSTRIDE,), jnp.float32),
      ],
      compiler_params=pltpu.CompilerParams(needs_layout_passes=False),
  )
  def k(rows_hbm, out_hbm, idx_v, deg_v):
    cid = lax.axis_index("c")
    sid = lax.axis_index("s")
    wid = sid * _NC + cid
    zeros16 = jnp.zeros((_LANES,), jnp.float32)
    ones16 = jnp.ones((_LANES,), jnp.float32)

    @pl.loop(0, _T * _TSTRIDE // _LANES)
    def _(i):
      deg_v[pl.ds(i * _LANES, _LANES)] = zeros16

    for t in range(_T):
      base_blk = t * (_E // _CHD) + wid * (_EPT_DEG // _CHD)
      off16 = jnp.full((_LANES,), t * _TSTRIDE, jnp.int32)
      for b in range(_EPT_DEG // _CHD):
        pltpu.sync_copy(rows_hbm.at[base_blk + b], idx_v)

        @pl.loop(0, _CHD // _LANES)
        def _(j):
          v = idx_v[pl.ds(j * _LANES, _LANES)] + off16
          plsc.addupdate_scatter(deg_v, [v], ones16)

    pltpu.sync_copy(deg_v, out_hbm.at[wid])

  return k(rows2d)


def _sc_scatter(edges3d, y0, y1):
  @functools.partial(
      pl.kernel,
      out_type=[
          jax.ShapeDtypeStruct((_N, _HALF), jnp.float32),
          jax.ShapeDtypeStruct((_N, _HALF), jnp.float32),
      ],
      mesh=_sc_mesh(),
      scratch_types=[
          pltpu.VMEM_SHARED((_N, _HALF), jnp.float32),
          pltpu.VMEM((2 * _KB, _CH), jnp.int32),
          pltpu.VMEM((2 * _KB, _CH), jnp.int32),
          [pltpu.VMEM((_CH, _HALF), jnp.float32) for _ in range(_KB)],
          [pltpu.VMEM((_CH, _HALF), jnp.float32) for _ in range(_KB)],
          pltpu.SemaphoreType.DMA,
          pltpu.SemaphoreType.DMA,
          pltpu.SemaphoreType.DMA,
      ],
      compiler_params=pltpu.CompilerParams(needs_layout_passes=False),
  )
  def k(edges_hbm, y0_hbm, y1_hbm, agg0_hbm, agg1_hbm,
        acc, idx_a, idx_b, gat_a, gat_b, gsem, ssem_a, ssem_b):
    cid = lax.axis_index("c")
    sid = lax.axis_index("s")
    last = _NS - 1
    tail = _N - last * _RPT

    def init_acc(y_hbm):
      @pl.when(sid < last)
      def _():
        pltpu.sync_copy(y_hbm.at[pl.ds(sid * _RPT, _RPT)],
                        acc.at[pl.ds(sid * _RPT, _RPT)])
      @pl.when(sid == last)
      def _():
        pltpu.sync_copy(y_hbm.at[pl.ds(last * _RPT, tail)],
                        acc.at[pl.ds(last * _RPT, tail)])

    def edge_loop(y_hbm):
      def process(si, idx, gat, ssem):
        blk = si * _NS + sid
        pltpu.sync_copy(edges_hbm.at[blk], idx)
        gd = [pltpu.async_copy(y_hbm.at[idx.at[j]], gat[j], gsem)
              for j in range(_KB)]
        for d in gd:
          d.wait()
        for j in range(_KB):
          pltpu.async_copy(gat[j], acc.at[idx.at[_KB + j]], ssem, add=True)

      def drain(gat, ssem):
        for j in range(_KB):
          pltpu.make_async_copy(y_hbm.at[pl.ds(0, _CH)], gat[j], ssem).wait()

      @pl.loop(0, (_NCPT - 1) // 2)
      def _(k):
        @pl.when(k > 0)
        def _():
          drain(gat_a, ssem_a)
        process(2 * k, idx_a, gat_a, ssem_a)
        @pl.when(k > 0)
        def _():
          drain(gat_b, ssem_b)
        process(2 * k + 1, idx_b, gat_b, ssem_b)

      drain(gat_a, ssem_a)
      process(_NCPT - 1, idx_a, gat_a, ssem_a)
      drain(gat_b, ssem_b)
      drain(gat_a, ssem_a)

    def copy_out(agg_hbm):
      @pl.when(sid < last)
      def _():
        pltpu.sync_copy(acc.at[pl.ds(sid * _RPT, _RPT)],
                        agg_hbm.at[pl.ds(sid * _RPT, _RPT)])
      @pl.when(sid == last)
      def _():
        pltpu.sync_copy(acc.at[pl.ds(last * _RPT, tail)],
                        agg_hbm.at[pl.ds(last * _RPT, tail)])

    @pl.when(cid == 0)
    def _():
      init_acc(y0_hbm)
    @pl.when(cid == 1)
    def _():
      init_acc(y1_hbm)
    plsc.subcore_barrier()
    @pl.when(cid == 0)
    def _():
      edge_loop(y0_hbm)
    @pl.when(cid == 1)
    def _():
      edge_loop(y1_hbm)
    plsc.subcore_barrier()
    @pl.when(cid == 0)
    def _():
      copy_out(agg0_hbm)
    @pl.when(cid == 1)
    def _():
      copy_out(agg1_hbm)

  return k(edges3d, y0, y1)


_R = 1000


def _tc_params():
  return pltpu.CompilerParams(dimension_semantics=("parallel",))


def _tc_dinv(deg_part):
  blk = 1280
  grid = (_T * _TSTRIDE) // blk

  def body(p_ref, o_ref):
    s = jnp.sum(p_ref[...], axis=0, keepdims=True) + 1.0
    o_ref[...] = lax.rsqrt(s)

  return pl.pallas_call(
      body,
      grid=(grid,),
      in_specs=[pl.BlockSpec((_NC * _NS, blk), lambda i: (0, i))],
      out_specs=pl.BlockSpec((1, blk), lambda i: (0, i)),
      out_shape=jax.ShapeDtypeStruct((1, _T * _TSTRIDE), jnp.float32),
      compiler_params=_tc_params(),
  )(deg_part)


def _tc_x1(feat, W1):
  def body(f_ref, w_ref, o_ref):
    o_ref[...] = jnp.dot(f_ref[...], w_ref[...],
                         preferred_element_type=jnp.float32)

  return pl.pallas_call(
      body,
      grid=(_N // _R,),
      in_specs=[
          pl.BlockSpec((_R, _DIN), lambda i: (i, 0)),
          pl.BlockSpec((_DIN, _DH), lambda i: (0, 0)),
      ],
      out_specs=pl.BlockSpec((_R, _DH), lambda i: (i, 0)),
      out_shape=jax.ShapeDtypeStruct((_N, _DH), jnp.float32),
      compiler_params=_tc_params(),
  )(feat, W1)


def _tc_prep(X1, dv):
  def body(x_ref, d_ref, y0_ref, y1_ref):
    y = x_ref[...] * d_ref[...]
    y0_ref[...] = y[:, :_HALF]
    y1_ref[...] = y[:, _HALF:]

  return pl.pallas_call(
      body,
      grid=(_N // _R,),
      in_specs=[
          pl.BlockSpec((_R, _DH), lambda i: (i, 0)),
          pl.BlockSpec((_R, 1), lambda i: (i, 0)),
      ],
      out_specs=[
          pl.BlockSpec((_R, _HALF), lambda i: (i, 0)),
          pl.BlockSpec((_R, _HALF), lambda i: (i, 0)),
      ],
      out_shape=[
          jax.ShapeDtypeStruct((_N, _HALF), jnp.float32),
          jax.ShapeDtypeStruct((_N, _HALF), jnp.float32),
      ],
      compiler_params=_tc_params(),
  )(X1, dv)


def _gru_math(h, hid, WihT, WhhT, bih, bhh):
  gi = jnp.dot(h, WihT, preferred_element_type=jnp.float32) + bih
  gh = jnp.dot(hid, WhhT, preferred_element_type=jnp.float32) + bhh
  r = jax.nn.sigmoid(gi[:, :_DH] + gh[:, :_DH])
  z = jax.nn.sigmoid(gi[:, _DH:2 * _DH] + gh[:, _DH:2 * _DH])
  n = jnp.tanh(gi[:, 2 * _DH:] + r * gh[:, 2 * _DH:])
  return (1.0 - z) * n + z * hid


def _tc_fused1(first, agg0, agg1, dv, hid, WihT, WhhT, bih, bhh, b, W2):
  def body(*refs):
    if first:
      (a0_ref, a1_ref, d_ref, wi_ref, wh_ref, bi_ref, bh_ref, b_ref, w2_ref,
       g_ref, z0_ref, z1_ref) = refs
    else:
      (a0_ref, a1_ref, d_ref, hid_ref, wi_ref, wh_ref, bi_ref, bh_ref, b_ref,
       w2_ref, g_ref, z0_ref, z1_ref) = refs
    dvb = d_ref[...]
    h = jnp.concatenate([a0_ref[...], a1_ref[...]], axis=1) * dvb + b_ref[...]
    hidv = h if first else hid_ref[...]
    g = _gru_math(h, hidv, wi_ref[...], wh_ref[...], bi_ref[...], bh_ref[...])
    g_ref[...] = g
    y2 = jnp.dot(g, w2_ref[...], preferred_element_type=jnp.float32) * dvb
    z0_ref[...] = y2[:, :_HALF]
    z1_ref[...] = y2[:, _HALF:]

  half_spec = pl.BlockSpec((_R, _HALF), lambda i: (i, 0))
  in_specs = [half_spec, half_spec, pl.BlockSpec((_R, 1), lambda i: (i, 0))]
  ins = [agg0, agg1, dv]
  if not first:
    in_specs.append(pl.BlockSpec((_R, _DH), lambda i: (i, 0)))
    ins.append(hid)
  in_specs += [
      pl.BlockSpec((_DH, 3 * _DH), lambda i: (0, 0)),
      pl.BlockSpec((_DH, 3 * _DH), lambda i: (0, 0)),
      pl.BlockSpec((1, 3 * _DH), lambda i: (0, 0)),
      pl.BlockSpec((1, 3 * _DH), lambda i: (0, 0)),
      pl.BlockSpec((1, _DH), lambda i: (0, 0)),
      pl.BlockSpec((_DH, _DH), lambda i: (0, 0)),
  ]
  ins += [WihT, WhhT, bih, bhh, b, W2]
  return pl.pallas_call(
      body,
      grid=(_N // _R,),
      in_specs=in_specs,
      out_specs=[
          pl.BlockSpec((_R, _DH), lambda i: (i, 0)),
          half_spec,
          half_spec,
      ],
      out_shape=[
          jax.ShapeDtypeStruct((_N, _DH), jnp.float32),
          jax.ShapeDtypeStruct((_N, _HALF), jnp.float32),
          jax.ShapeDtypeStruct((_N, _HALF), jnp.float32),
      ],
      compiler_params=_tc_params(),
  )(*ins)


def _tc_fused2(first, last, agg0, agg1, dv, hid, WihT, WhhT, bih, bhh, b,
               X1, dvn):
  def body(*refs):
    refs = list(refs)
    a0_ref = refs.pop(0)
    a1_ref = refs.pop(0)
    d_ref = refs.pop(0)
    hid_ref = None if first else refs.pop(0)
    wi_ref = refs.pop(0)
    wh_ref = refs.pop(0)
    bi_ref = refs.pop(0)
    bh_ref = refs.pop(0)
    b_ref = refs.pop(0)
    if not last:
      x1_ref = refs.pop(0)
      dn_ref = refs.pop(0)
    g_ref = refs.pop(0)
    dvb = d_ref[...]
    h = jnp.concatenate([a0_ref[...], a1_ref[...]], axis=1) * dvb + b_ref[...]
    hidv = h if first else hid_ref[...]
    g = _gru_math(h, hidv, wi_ref[...], wh_ref[...], bi_ref[...], bh_ref[...])
    g_ref[...] = g
    if not last:
      y0_ref = refs.pop(0)
      y1_ref = refs.pop(0)
      yn = x1_ref[...] * dn_ref[...]
      y0_ref[...] = yn[:, :_HALF]
      y1_ref[...] = yn[:, _HALF:]

  half_spec = pl.BlockSpec((_R, _HALF), lambda i: (i, 0))
  dv_spec = pl.BlockSpec((_R, 1), lambda i: (i, 0))
  in_specs = [half_spec, half_spec, dv_spec]
  ins = [agg0, agg1, dv]
  if not first:
    in_specs.append(pl.BlockSpec((_R, _DH), lambda i: (i, 0)))
    ins.append(hid)
  in_specs += [
      pl.BlockSpec((_DH, 3 * _DH), lambda i: (0, 0)),
      pl.BlockSpec((_DH, 3 * _DH), lambda i: (0, 0)),
      pl.BlockSpec((1, 3 * _DH), lambda i: (0, 0)),
      pl.BlockSpec((1, 3 * _DH), lambda i: (0, 0)),
      pl.BlockSpec((1, _DH), lambda i: (0, 0)),
  ]
  ins += [WihT, WhhT, bih, bhh, b]
  if not last:
    in_specs += [pl.BlockSpec((_R, _DH), lambda i: (i, 0)), dv_spec]
    ins += [X1, dvn]
  out_specs = [pl.BlockSpec((_R, _DH), lambda i: (i, 0))]
  out_shape = [jax.ShapeDtypeStruct((_N, _DH), jnp.float32)]
  if not last:
    out_specs += [half_spec, half_spec]
    out_shape += [
        jax.ShapeDtypeStruct((_N, _HALF), jnp.float32),
        jax.ShapeDtypeStruct((_N, _HALF), jnp.float32),
    ]
  res = pl.pallas_call(
      body,
      grid=(_N // _R,),
      in_specs=in_specs,
      out_specs=out_specs,
      out_shape=out_shape,
      compiler_params=_tc_params(),
  )(*ins)
  return res[0] if last else res


def kernel(feat, graphs, W1, b1, W2, b2, Wih1, Whh1, bih1, bhh1,
           Wih2, Whh2, bih2, bhh2):
  rows_all = graphs[:, 0, :].reshape(_T * _E // _CHD, _CHD)
  deg_part = _sc_degree(rows_all)
  dinv = _tc_dinv(deg_part).reshape(_T, _TSTRIDE)

  X1 = _tc_x1(feat, W1)
  Wih1T, Whh1T = Wih1.T, Whh1.T
  Wih2T, Whh2T = Wih2.T, Whh2.T
  bih1r, bhh1r = bih1.reshape(1, -1), bhh1.reshape(1, -1)
  bih2r, bhh2r = bih2.reshape(1, -1), bhh2.reshape(1, -1)
  b1r, b2r = b1.reshape(1, -1), b2.reshape(1, -1)

  y0, y1h = _tc_prep(X1, dinv[0].reshape(_TSTRIDE, 1))
  g1 = None
  g2 = None
  outs = []
  for t in range(_T):
    r3 = graphs[t, 0].reshape(_NSUP, _KB, _CH)
    c3 = graphs[t, 1].reshape(_NSUP, _KB, _CH)
    edges3d = jnp.concatenate([r3, c3], axis=1)
    dv = dinv[t].reshape(_TSTRIDE, 1)

    agg0, agg1 = _sc_scatter(edges3d, y0, y1h)
    g1, z0, z1 = _tc_fused1(t == 0, agg0, agg1, dv, g1,
                            Wih1T, Whh1T, bih1r, bhh1r, b1r, W2)

    agg20, agg21 = _sc_scatter(edges3d, z0, z1)
    if t < _T - 1:
      dvn = dinv[t + 1].reshape(_TSTRIDE, 1)
      g2, y0, y1h = _tc_fused2(t == 0, False, agg20, agg21, dv, g2,
                               Wih2T, Whh2T, bih2r, bhh2r, b2r, X1, dvn)
    else:
      g2 = _tc_fused2(t == 0, True, agg20, agg21, dv, g2,
                      Wih2T, Whh2T, bih2r, bhh2r, b2r, None, None)
    outs.append(g2)
  return jnp.stack(outs)


# speedup vs baseline: 15.1293x; 1.0648x over previous
import functools

import jax
import jax.numpy as jnp
from jax import lax
from jax.experimental import pallas as pl
from jax.experimental.pallas import tpu as pltpu
from jax.experimental.pallas import tpu_sc as plsc

_N = 10000
_E = 320000
_T = 3
_DIN = 128
_DH = 256
_HALF = 128

_NC = 2
_NS = 16
_LANES = 16

_TSTRIDE = 10240
_CHD = 2000
_EPT_DEG = _E // (_NC * _NS)

_CH = 80
_KB = 2
_SUP = _CH * _KB
_NSUP = _E // _SUP
_NCPT = _NSUP // _NS
_RPT = 640


def _sc_mesh():
  return plsc.VectorSubcoreMesh(
      core_axis_name="c", subcore_axis_name="s",
      num_cores=_NC, num_subcores=_NS)


def _sc_degree(rows2d):
  @functools.partial(
      pl.kernel,
      out_type=jax.ShapeDtypeStruct((_NC * _NS, _T * _TSTRIDE), jnp.float32),
      mesh=_sc_mesh(),
      scratch_types=[
          pltpu.VMEM((_CHD,), jnp.int32),
          pltpu.VMEM((_T * _TSTRIDE,), jnp.float32),
      ],
      compiler_params=pltpu.CompilerParams(needs_layout_passes=False),
  )
  def k(rows_hbm, out_hbm, idx_v, deg_v):
    cid = lax.axis_index("c")
    sid = lax.axis_index("s")
    wid = sid * _NC + cid
    zeros16 = jnp.zeros((_LANES,), jnp.float32)
    ones16 = jnp.ones((_LANES,), jnp.float32)

    @pl.loop(0, _T * _TSTRIDE // _LANES)
    def _(i):
      deg_v[pl.ds(i * _LANES, _LANES)] = zeros16

    for t in range(_T):
      base_blk = t * (_E // _CHD) + wid * (_EPT_DEG // _CHD)
      off16 = jnp.full((_LANES,), t * _TSTRIDE, jnp.int32)
      for b in range(_EPT_DEG // _CHD):
        pltpu.sync_copy(rows_hbm.at[base_blk + b], idx_v)

        @pl.loop(0, _CHD // _LANES)
        def _(j):
          v = idx_v[pl.ds(j * _LANES, _LANES)] + off16
          plsc.addupdate_scatter(deg_v, [v], ones16)

    pltpu.sync_copy(deg_v, out_hbm.at[wid])

  return k(rows2d)


def _sc_scatter(rows3d, cols3d, y0, y1):
  @functools.partial(
      pl.kernel,
      out_type=[
          jax.ShapeDtypeStruct((_N, _HALF), jnp.float32),
          jax.ShapeDtypeStruct((_N, _HALF), jnp.float32),
      ],
      mesh=_sc_mesh(),
      scratch_types=[
          pltpu.VMEM_SHARED((_N, _HALF), jnp.float32),
          pltpu.VMEM((_KB, _CH), jnp.int32),
          pltpu.VMEM((_KB, _CH), jnp.int32),
          pltpu.VMEM((_KB, _CH), jnp.int32),
          pltpu.VMEM((_KB, _CH), jnp.int32),
          [pltpu.VMEM((_CH, _HALF), jnp.float32) for _ in range(_KB)],
          [pltpu.VMEM((_CH, _HALF), jnp.float32) for _ in range(_KB)],
          pltpu.SemaphoreType.DMA,
          pltpu.SemaphoreType.DMA,
          pltpu.SemaphoreType.DMA,
          pltpu.SemaphoreType.DMA,
      ],
      compiler_params=pltpu.CompilerParams(needs_layout_passes=False),
  )
  def k(rows_hbm, cols_hbm, y0_hbm, y1_hbm, agg0_hbm, agg1_hbm,
        acc, idx_ga, idx_gb, idx_sa, idx_sb, gat_a, gat_b,
        gsem_a, gsem_b, ssem_a, ssem_b):
    cid = lax.axis_index("c")
    sid = lax.axis_index("s")
    last = _NS - 1
    tail = _N - last * _RPT

    def init_acc(y_hbm):
      @pl.when(sid < last)
      def _():
        pltpu.sync_copy(y_hbm.at[pl.ds(sid * _RPT, _RPT)],
                        acc.at[pl.ds(sid * _RPT, _RPT)])
      @pl.when(sid == last)
      def _():
        pltpu.sync_copy(y_hbm.at[pl.ds(last * _RPT, tail)],
                        acc.at[pl.ds(last * _RPT, tail)])

    def edge_loop(y_hbm):
      def load_fire(si, idx_g, idx_s, gat, gsem):
        blk = si * _NS + sid
        pltpu.sync_copy(rows_hbm.at[blk], idx_g)
        pltpu.sync_copy(cols_hbm.at[blk], idx_s)
        for j in range(_KB):
          pltpu.async_copy(y_hbm.at[idx_g.at[j]], gat[j], gsem)

      def fire_scatter(idx_s, gat, ssem):
        for j in range(_KB):
          pltpu.async_copy(gat[j], acc.at[idx_s.at[j]], ssem, add=True)

      def drain(gat, sem):
        for j in range(_KB):
          pltpu.make_async_copy(y_hbm.at[pl.ds(0, _CH)], gat[j], sem).wait()

      load_fire(0, idx_ga, idx_sa, gat_a, gsem_a)

      @pl.loop(0, (_NCPT + 1) // 2)
      def _(k):
        drain(gat_a, gsem_a)
        fire_scatter(idx_sa, gat_a, ssem_a)
        @pl.when(k > 0)
        def _():
          drain(gat_b, ssem_b)
        @pl.when(2 * k + 1 < _NCPT)
        def _():
          load_fire(2 * k + 1, idx_gb, idx_sb, gat_b, gsem_b)
        drain(gat_a, ssem_a)
        @pl.when(2 * k + 2 < _NCPT)
        def _():
          load_fire(2 * k + 2, idx_ga, idx_sa, gat_a, gsem_a)
        @pl.when(2 * k + 1 < _NCPT)
        def _():
          drain(gat_b, gsem_b)
          fire_scatter(idx_sb, gat_b, ssem_b)

    def copy_out(agg_hbm):
      @pl.when(sid < last)
      def _():
        pltpu.sync_copy(acc.at[pl.ds(sid * _RPT, _RPT)],
                        agg_hbm.at[pl.ds(sid * _RPT, _RPT)])
      @pl.when(sid == last)
      def _():
        pltpu.sync_copy(acc.at[pl.ds(last * _RPT, tail)],
                        agg_hbm.at[pl.ds(last * _RPT, tail)])

    @pl.when(cid == 0)
    def _():
      init_acc(y0_hbm)
    @pl.when(cid == 1)
    def _():
      init_acc(y1_hbm)
    plsc.subcore_barrier()
    @pl.when(cid == 0)
    def _():
      edge_loop(y0_hbm)
    @pl.when(cid == 1)
    def _():
      edge_loop(y1_hbm)
    plsc.subcore_barrier()
    @pl.when(cid == 0)
    def _():
      copy_out(agg0_hbm)
    @pl.when(cid == 1)
    def _():
      copy_out(agg1_hbm)

  return k(rows3d, cols3d, y0, y1)


_R = 1000


def _tc_params():
  return pltpu.CompilerParams(dimension_semantics=("parallel",))


def _tc_dinv(deg_part):
  blk = 1280
  grid = (_T * _TSTRIDE) // blk

  def body(p_ref, o_ref):
    s = jnp.sum(p_ref[...], axis=0, keepdims=True) + 1.0
    o_ref[...] = lax.rsqrt(s)

  return pl.pallas_call(
      body,
      grid=(grid,),
      in_specs=[pl.BlockSpec((_NC * _NS, blk), lambda i: (0, i))],
      out_specs=pl.BlockSpec((1, blk), lambda i: (0, i)),
      out_shape=jax.ShapeDtypeStruct((1, _T * _TSTRIDE), jnp.float32),
      compiler_params=_tc_params(),
  )(deg_part)


def _tc_x1(feat, W1):
  def body(f_ref, w_ref, o_ref):
    o_ref[...] = jnp.dot(f_ref[...], w_ref[...],
                         preferred_element_type=jnp.float32)

  return pl.pallas_call(
      body,
      grid=(_N // _R,),
      in_specs=[
          pl.BlockSpec((_R, _DIN), lambda i: (i, 0)),
          pl.BlockSpec((_DIN, _DH), lambda i: (0, 0)),
      ],
      out_specs=pl.BlockSpec((_R, _DH), lambda i: (i, 0)),
      out_shape=jax.ShapeDtypeStruct((_N, _DH), jnp.float32),
      compiler_params=_tc_params(),
  )(feat, W1)


def _tc_prep(X1, dv):
  def body(x_ref, d_ref, y0_ref, y1_ref):
    y = x_ref[...] * d_ref[...]
    y0_ref[...] = y[:, :_HALF]
    y1_ref[...] = y[:, _HALF:]

  return pl.pallas_call(
      body,
      grid=(_N // _R,),
      in_specs=[
          pl.BlockSpec((_R, _DH), lambda i: (i, 0)),
          pl.BlockSpec((_R, 1), lambda i: (i, 0)),
      ],
      out_specs=[
          pl.BlockSpec((_R, _HALF), lambda i: (i, 0)),
          pl.BlockSpec((_R, _HALF), lambda i: (i, 0)),
      ],
      out_shape=[
          jax.ShapeDtypeStruct((_N, _HALF), jnp.float32),
          jax.ShapeDtypeStruct((_N, _HALF), jnp.float32),
      ],
      compiler_params=_tc_params(),
  )(X1, dv)


def _gru_math(h, hid, WihT, WhhT, bih, bhh):
  gi = jnp.dot(h, WihT, preferred_element_type=jnp.float32) + bih
  gh = jnp.dot(hid, WhhT, preferred_element_type=jnp.float32) + bhh
  r = jax.nn.sigmoid(gi[:, :_DH] + gh[:, :_DH])
  z = jax.nn.sigmoid(gi[:, _DH:2 * _DH] + gh[:, _DH:2 * _DH])
  n = jnp.tanh(gi[:, 2 * _DH:] + r * gh[:, 2 * _DH:])
  return (1.0 - z) * n + z * hid


def _tc_fused1(first, agg0, agg1, dv, hid, WihT, WhhT, bih, bhh, b, W2):
  def body(*refs):
    if first:
      (a0_ref, a1_ref, d_ref, wi_ref, wh_ref, bi_ref, bh_ref, b_ref, w2_ref,
       g_ref, z0_ref, z1_ref) = refs
    else:
      (a0_ref, a1_ref, d_ref, hid_ref, wi_ref, wh_ref, bi_ref, bh_ref, b_ref,
       w2_ref, g_ref, z0_ref, z1_ref) = refs
    dvb = d_ref[...]
    h = jnp.concatenate([a0_ref[...], a1_ref[...]], axis=1) * dvb + b_ref[...]
    hidv = h if first else hid_ref[...]
    g = _gru_math(h, hidv, wi_ref[...], wh_ref[...], bi_ref[...], bh_ref[...])
    g_ref[...] = g
    y2 = jnp.dot(g, w2_ref[...], preferred_element_type=jnp.float32) * dvb
    z0_ref[...] = y2[:, :_HALF]
    z1_ref[...] = y2[:, _HALF:]

  half_spec = pl.BlockSpec((_R, _HALF), lambda i: (i, 0))
  in_specs = [half_spec, half_spec, pl.BlockSpec((_R, 1), lambda i: (i, 0))]
  ins = [agg0, agg1, dv]
  if not first:
    in_specs.append(pl.BlockSpec((_R, _DH), lambda i: (i, 0)))
    ins.append(hid)
  in_specs += [
      pl.BlockSpec((_DH, 3 * _DH), lambda i: (0, 0)),
      pl.BlockSpec((_DH, 3 * _DH), lambda i: (0, 0)),
      pl.BlockSpec((1, 3 * _DH), lambda i: (0, 0)),
      pl.BlockSpec((1, 3 * _DH), lambda i: (0, 0)),
      pl.BlockSpec((1, _DH), lambda i: (0, 0)),
      pl.BlockSpec((_DH, _DH), lambda i: (0, 0)),
  ]
  ins += [WihT, WhhT, bih, bhh, b, W2]
  return pl.pallas_call(
      body,
      grid=(_N // _R,),
      in_specs=in_specs,
      out_specs=[
          pl.BlockSpec((_R, _DH), lambda i: (i, 0)),
          half_spec,
          half_spec,
      ],
      out_shape=[
          jax.ShapeDtypeStruct((_N, _DH), jnp.float32),
          jax.ShapeDtypeStruct((_N, _HALF), jnp.float32),
          jax.ShapeDtypeStruct((_N, _HALF), jnp.float32),
      ],
      compiler_params=_tc_params(),
  )(*ins)


def _tc_fused2(first, last, agg0, agg1, dv, hid, WihT, WhhT, bih, bhh, b,
               X1, dvn):
  def body(*refs):
    refs = list(refs)
    a0_ref = refs.pop(0)
    a1_ref = refs.pop(0)
    d_ref = refs.pop(0)
    hid_ref = None if first else refs.pop(0)
    wi_ref = refs.pop(0)
    wh_ref = refs.pop(0)
    bi_ref = refs.pop(0)
    bh_ref = refs.pop(0)
    b_ref = refs.pop(0)
    if not last:
      x1_ref = refs.pop(0)
      dn_ref = refs.pop(0)
    g_ref = refs.pop(0)
    dvb = d_ref[...]
    h = jnp.concatenate([a0_ref[...], a1_ref[...]], axis=1) * dvb + b_ref[...]
    hidv = h if first else hid_ref[...]
    g = _gru_math(h, hidv, wi_ref[...], wh_ref[...], bi_ref[...], bh_ref[...])
    g_ref[...] = g
    if not last:
      y0_ref = refs.pop(0)
      y1_ref = refs.pop(0)
      yn = x1_ref[...] * dn_ref[...]
      y0_ref[...] = yn[:, :_HALF]
      y1_ref[...] = yn[:, _HALF:]

  half_spec = pl.BlockSpec((_R, _HALF), lambda i: (i, 0))
  dv_spec = pl.BlockSpec((_R, 1), lambda i: (i, 0))
  in_specs = [half_spec, half_spec, dv_spec]
  ins = [agg0, agg1, dv]
  if not first:
    in_specs.append(pl.BlockSpec((_R, _DH), lambda i: (i, 0)))
    ins.append(hid)
  in_specs += [
      pl.BlockSpec((_DH, 3 * _DH), lambda i: (0, 0)),
      pl.BlockSpec((_DH, 3 * _DH), lambda i: (0, 0)),
      pl.BlockSpec((1, 3 * _DH), lambda i: (0, 0)),
      pl.BlockSpec((1, 3 * _DH), lambda i: (0, 0)),
      pl.BlockSpec((1, _DH), lambda i: (0, 0)),
  ]
  ins += [WihT, WhhT, bih, bhh, b]
  if not last:
    in_specs += [pl.BlockSpec((_R, _DH), lambda i: (i, 0)), dv_spec]
    ins += [X1, dvn]
  out_specs = [pl.BlockSpec((_R, _DH), lambda i: (i, 0))]
  out_shape = [jax.ShapeDtypeStruct((_N, _DH), jnp.float32)]
  if not last:
    out_specs += [half_spec, half_spec]
    out_shape += [
        jax.ShapeDtypeStruct((_N, _HALF), jnp.float32),
        jax.ShapeDtypeStruct((_N, _HALF), jnp.float32),
    ]
  res = pl.pallas_call(
      body,
      grid=(_N // _R,),
      in_specs=in_specs,
      out_specs=out_specs,
      out_shape=out_shape,
      compiler_params=_tc_params(),
  )(*ins)
  return res[0] if last else res


def kernel(feat, graphs, W1, b1, W2, b2, Wih1, Whh1, bih1, bhh1,
           Wih2, Whh2, bih2, bhh2):
  rows_all = graphs[:, 0, :].reshape(_T * _E // _CHD, _CHD)
  deg_part = _sc_degree(rows_all)
  dinv = _tc_dinv(deg_part).reshape(_T, _TSTRIDE)

  X1 = _tc_x1(feat, W1)
  Wih1T, Whh1T = Wih1.T, Whh1.T
  Wih2T, Whh2T = Wih2.T, Whh2.T
  bih1r, bhh1r = bih1.reshape(1, -1), bhh1.reshape(1, -1)
  bih2r, bhh2r = bih2.reshape(1, -1), bhh2.reshape(1, -1)
  b1r, b2r = b1.reshape(1, -1), b2.reshape(1, -1)

  y0, y1h = _tc_prep(X1, dinv[0].reshape(_TSTRIDE, 1))
  g1 = None
  g2 = None
  outs = []
  for t in range(_T):
    r3 = graphs[t, 0].reshape(_NSUP, _KB, _CH)
    c3 = graphs[t, 1].reshape(_NSUP, _KB, _CH)
    dv = dinv[t].reshape(_TSTRIDE, 1)

    agg0, agg1 = _sc_scatter(r3, c3, y0, y1h)
    g1, z0, z1 = _tc_fused1(t == 0, agg0, agg1, dv, g1,
                            Wih1T, Whh1T, bih1r, bhh1r, b1r, W2)

    agg20, agg21 = _sc_scatter(r3, c3, z0, z1)
    if t < _T - 1:
      dvn = dinv[t + 1].reshape(_TSTRIDE, 1)
      g2, y0, y1h = _tc_fused2(t == 0, False, agg20, agg21, dv, g2,
                               Wih2T, Whh2T, bih2r, bhh2r, b2r, X1, dvn)
    else:
      g2 = _tc_fused2(t == 0, True, agg20, agg21, dv, g2,
                      Wih2T, Whh2T, bih2r, bhh2r, b2r, None, None)
    outs.append(g2)
  return jnp.stack(outs)


# speedup vs baseline: 15.8717x; 1.0491x over previous
import functools

import jax
import jax.numpy as jnp
from jax import lax
from jax.experimental import pallas as pl
from jax.experimental.pallas import tpu as pltpu
from jax.experimental.pallas import tpu_sc as plsc

_N = 10000
_E = 320000
_T = 3
_DIN = 128
_DH = 256
_HALF = 128

_NC = 2
_NS = 16
_LANES = 16

_TSTRIDE = 10240
_CHD = 2000
_EPT_DEG = _E // (_NC * _NS)

_CH = 80
_KB = 2
_SUP = _CH * _KB
_NSUP = _E // _SUP
_NCPT = _NSUP // _NS
_MBLK = 5
_RPT = 640


def _sc_mesh():
  return plsc.VectorSubcoreMesh(
      core_axis_name="c", subcore_axis_name="s",
      num_cores=_NC, num_subcores=_NS)


def _sc_degree(rows2d):
  @functools.partial(
      pl.kernel,
      out_type=jax.ShapeDtypeStruct((_NC * _NS, _T * _TSTRIDE), jnp.float32),
      mesh=_sc_mesh(),
      scratch_types=[
          pltpu.VMEM((_CHD,), jnp.int32),
          pltpu.VMEM((_T * _TSTRIDE,), jnp.float32),
      ],
      compiler_params=pltpu.CompilerParams(needs_layout_passes=False),
  )
  def k(rows_hbm, out_hbm, idx_v, deg_v):
    cid = lax.axis_index("c")
    sid = lax.axis_index("s")
    wid = sid * _NC + cid
    zeros16 = jnp.zeros((_LANES,), jnp.float32)
    ones16 = jnp.ones((_LANES,), jnp.float32)

    @pl.loop(0, _T * _TSTRIDE // _LANES)
    def _(i):
      deg_v[pl.ds(i * _LANES, _LANES)] = zeros16

    for t in range(_T):
      base_blk = t * (_E // _CHD) + wid * (_EPT_DEG // _CHD)
      off16 = jnp.full((_LANES,), t * _TSTRIDE, jnp.int32)
      for b in range(_EPT_DEG // _CHD):
        pltpu.sync_copy(rows_hbm.at[base_blk + b], idx_v)

        @pl.loop(0, _CHD // _LANES)
        def _(j):
          v = idx_v[pl.ds(j * _LANES, _LANES)] + off16
          plsc.addupdate_scatter(deg_v, [v], ones16)

    pltpu.sync_copy(deg_v, out_hbm.at[wid])

  return k(rows2d)


def _sc_scatter(rows3d, cols3d, y0, y1):
  @functools.partial(
      pl.kernel,
      out_type=[
          jax.ShapeDtypeStruct((_N, _HALF), jnp.float32),
          jax.ShapeDtypeStruct((_N, _HALF), jnp.float32),
      ],
      mesh=_sc_mesh(),
      scratch_types=[
          pltpu.VMEM_SHARED((_N, _HALF), jnp.float32),
          pltpu.VMEM((_MBLK, _KB, _CH), jnp.int32),
          pltpu.VMEM((_MBLK, _KB, _CH), jnp.int32),
          [pltpu.VMEM((_CH, _HALF), jnp.float32) for _ in range(_KB)],
          [pltpu.VMEM((_CH, _HALF), jnp.float32) for _ in range(_KB)],
          pltpu.SemaphoreType.DMA,
          pltpu.SemaphoreType.DMA,
          pltpu.SemaphoreType.DMA,
          pltpu.SemaphoreType.DMA,
      ],
      compiler_params=pltpu.CompilerParams(needs_layout_passes=False),
  )
  def k(rows_hbm, cols_hbm, y0_hbm, y1_hbm, agg0_hbm, agg1_hbm,
        acc, idx_r, idx_c, gat_a, gat_b,
        gsem_a, gsem_b, ssem_a, ssem_b):
    cid = lax.axis_index("c")
    sid = lax.axis_index("s")
    last = _NS - 1
    tail = _N - last * _RPT

    def init_acc(y_hbm):
      @pl.when(sid < last)
      def _():
        pltpu.sync_copy(y_hbm.at[pl.ds(sid * _RPT, _RPT)],
                        acc.at[pl.ds(sid * _RPT, _RPT)])
      @pl.when(sid == last)
      def _():
        pltpu.sync_copy(y_hbm.at[pl.ds(last * _RPT, tail)],
                        acc.at[pl.ds(last * _RPT, tail)])

    def edge_loop(y_hbm):
      def fire_gather(m, gat, gsem):
        for j in range(_KB):
          pltpu.async_copy(y_hbm.at[idx_r.at[m, j]], gat[j], gsem)

      def fire_scatter(m, gat, ssem):
        for j in range(_KB):
          pltpu.async_copy(gat[j], acc.at[idx_c.at[m, j]], ssem, add=True)

      def drain(gat, sem):
        for j in range(_KB):
          pltpu.make_async_copy(y_hbm.at[pl.ds(0, _CH)], gat[j], sem).wait()

      @pl.loop(0, _NCPT // _MBLK)
      def _(bi):
        base = sid * _NCPT + bi * _MBLK
        pltpu.sync_copy(rows_hbm.at[pl.ds(base, _MBLK)], idx_r)
        pltpu.sync_copy(cols_hbm.at[pl.ds(base, _MBLK)], idx_c)
        fire_gather(0, gat_a, gsem_a)

        @pl.loop(0, (_MBLK + 1) // 2)
        def _(k):
          drain(gat_a, gsem_a)
          fire_scatter(2 * k, gat_a, ssem_a)
          @pl.when(k > 0)
          def _():
            drain(gat_b, ssem_b)
          @pl.when(2 * k + 1 < _MBLK)
          def _():
            fire_gather(2 * k + 1, gat_b, gsem_b)
          drain(gat_a, ssem_a)
          @pl.when(2 * k + 2 < _MBLK)
          def _():
            fire_gather(2 * k + 2, gat_a, gsem_a)
          @pl.when(2 * k + 1 < _MBLK)
          def _():
            drain(gat_b, gsem_b)
            fire_scatter(2 * k + 1, gat_b, ssem_b)

    def copy_out(agg_hbm):
      @pl.when(sid < last)
      def _():
        pltpu.sync_copy(acc.at[pl.ds(sid * _RPT, _RPT)],
                        agg_hbm.at[pl.ds(sid * _RPT, _RPT)])
      @pl.when(sid == last)
      def _():
        pltpu.sync_copy(acc.at[pl.ds(last * _RPT, tail)],
                        agg_hbm.at[pl.ds(last * _RPT, tail)])

    @pl.when(cid == 0)
    def _():
      init_acc(y0_hbm)
    @pl.when(cid == 1)
    def _():
      init_acc(y1_hbm)
    plsc.subcore_barrier()
    @pl.when(cid == 0)
    def _():
      edge_loop(y0_hbm)
    @pl.when(cid == 1)
    def _():
      edge_loop(y1_hbm)
    plsc.subcore_barrier()
    @pl.when(cid == 0)
    def _():
      copy_out(agg0_hbm)
    @pl.when(cid == 1)
    def _():
      copy_out(agg1_hbm)

  return k(rows3d, cols3d, y0, y1)


_R = 1000


def _tc_params():
  return pltpu.CompilerParams(dimension_semantics=("parallel",))


def _tc_dinv(deg_part):
  blk = 1280
  grid = (_T * _TSTRIDE) // blk

  def body(p_ref, o_ref):
    s = jnp.sum(p_ref[...], axis=0, keepdims=True) + 1.0
    o_ref[...] = lax.rsqrt(s)

  return pl.pallas_call(
      body,
      grid=(grid,),
      in_specs=[pl.BlockSpec((_NC * _NS, blk), lambda i: (0, i))],
      out_specs=pl.BlockSpec((1, blk), lambda i: (0, i)),
      out_shape=jax.ShapeDtypeStruct((1, _T * _TSTRIDE), jnp.float32),
      compiler_params=_tc_params(),
  )(deg_part)


def _tc_x1(feat, W1):
  def body(f_ref, w_ref, o_ref):
    o_ref[...] = jnp.dot(f_ref[...], w_ref[...],
                         preferred_element_type=jnp.float32)

  return pl.pallas_call(
      body,
      grid=(_N // _R,),
      in_specs=[
          pl.BlockSpec((_R, _DIN), lambda i: (i, 0)),
          pl.BlockSpec((_DIN, _DH), lambda i: (0, 0)),
      ],
      out_specs=pl.BlockSpec((_R, _DH), lambda i: (i, 0)),
      out_shape=jax.ShapeDtypeStruct((_N, _DH), jnp.float32),
      compiler_params=_tc_params(),
  )(feat, W1)


def _tc_prep(X1, dv):
  def body(x_ref, d_ref, y0_ref, y1_ref):
    y = x_ref[...] * d_ref[...]
    y0_ref[...] = y[:, :_HALF]
    y1_ref[...] = y[:, _HALF:]

  return pl.pallas_call(
      body,
      grid=(_N // _R,),
      in_specs=[
          pl.BlockSpec((_R, _DH), lambda i: (i, 0)),
          pl.BlockSpec((_R, 1), lambda i: (i, 0)),
      ],
      out_specs=[
          pl.BlockSpec((_R, _HALF), lambda i: (i, 0)),
          pl.BlockSpec((_R, _HALF), lambda i: (i, 0)),
      ],
      out_shape=[
          jax.ShapeDtypeStruct((_N, _HALF), jnp.float32),
          jax.ShapeDtypeStruct((_N, _HALF), jnp.float32),
      ],
      compiler_params=_tc_params(),
  )(X1, dv)


def _gru_math(h, hid, WihT, WhhT, bih, bhh):
  gi = jnp.dot(h, WihT, preferred_element_type=jnp.float32) + bih
  gh = jnp.dot(hid, WhhT, preferred_element_type=jnp.float32) + bhh
  r = jax.nn.sigmoid(gi[:, :_DH] + gh[:, :_DH])
  z = jax.nn.sigmoid(gi[:, _DH:2 * _DH] + gh[:, _DH:2 * _DH])
  n = jnp.tanh(gi[:, 2 * _DH:] + r * gh[:, 2 * _DH:])
  return (1.0 - z) * n + z * hid


def _tc_fused1(first, agg0, agg1, dv, hid, WihT, WhhT, bih, bhh, b, W2):
  def body(*refs):
    if first:
      (a0_ref, a1_ref, d_ref, wi_ref, wh_ref, bi_ref, bh_ref, b_ref, w2_ref,
       g_ref, z0_ref, z1_ref) = refs
    else:
      (a0_ref, a1_ref, d_ref, hid_ref, wi_ref, wh_ref, bi_ref, bh_ref, b_ref,
       w2_ref, g_ref, z0_ref, z1_ref) = refs
    dvb = d_ref[...]
    h = jnp.concatenate([a0_ref[...], a1_ref[...]], axis=1) * dvb + b_ref[...]
    hidv = h if first else hid_ref[...]
    g = _gru_math(h, hidv, wi_ref[...], wh_ref[...], bi_ref[...], bh_ref[...])
    g_ref[...] = g
    y2 = jnp.dot(g, w2_ref[...], preferred_element_type=jnp.float32) * dvb
    z0_ref[...] = y2[:, :_HALF]
    z1_ref[...] = y2[:, _HALF:]

  half_spec = pl.BlockSpec((_R, _HALF), lambda i: (i, 0))
  in_specs = [half_spec, half_spec, pl.BlockSpec((_R, 1), lambda i: (i, 0))]
  ins = [agg0, agg1, dv]
  if not first:
    in_specs.append(pl.BlockSpec((_R, _DH), lambda i: (i, 0)))
    ins.append(hid)
  in_specs += [
      pl.BlockSpec((_DH, 3 * _DH), lambda i: (0, 0)),
      pl.BlockSpec((_DH, 3 * _DH), lambda i: (0, 0)),
      pl.BlockSpec((1, 3 * _DH), lambda i: (0, 0)),
      pl.BlockSpec((1, 3 * _DH), lambda i: (0, 0)),
      pl.BlockSpec((1, _DH), lambda i: (0, 0)),
      pl.BlockSpec((_DH, _DH), lambda i: (0, 0)),
  ]
  ins += [WihT, WhhT, bih, bhh, b, W2]
  return pl.pallas_call(
      body,
      grid=(_N // _R,),
      in_specs=in_specs,
      out_specs=[
          pl.BlockSpec((_R, _DH), lambda i: (i, 0)),
          half_spec,
          half_spec,
      ],
      out_shape=[
          jax.ShapeDtypeStruct((_N, _DH), jnp.float32),
          jax.ShapeDtypeStruct((_N, _HALF), jnp.float32),
          jax.ShapeDtypeStruct((_N, _HALF), jnp.float32),
      ],
      compiler_params=_tc_params(),
  )(*ins)


def _tc_fused2(first, last, agg0, agg1, dv, hid, WihT, WhhT, bih, bhh, b,
               X1, dvn):
  def body(*refs):
    refs = list(refs)
    a0_ref = refs.pop(0)
    a1_ref = refs.pop(0)
    d_ref = refs.pop(0)
    hid_ref = None if first else refs.pop(0)
    wi_ref = refs.pop(0)
    wh_ref = refs.pop(0)
    bi_ref = refs.pop(0)
    bh_ref = refs.pop(0)
    b_ref = refs.pop(0)
    if not last:
      x1_ref = refs.pop(0)
      dn_ref = refs.pop(0)
    g_ref = refs.pop(0)
    dvb = d_ref[...]
    h = jnp.concatenate([a0_ref[...], a1_ref[...]], axis=1) * dvb + b_ref[...]
    hidv = h if first else hid_ref[...]
    g = _gru_math(h, hidv, wi_ref[...], wh_ref[...], bi_ref[...], bh_ref[...])
    g_ref[...] = g
    if not last:
      y0_ref = refs.pop(0)
      y1_ref = refs.pop(0)
      yn = x1_ref[...] * dn_ref[...]
      y0_ref[...] = yn[:, :_HALF]
      y1_ref[...] = yn[:, _HALF:]

  half_spec = pl.BlockSpec((_R, _HALF), lambda i: (i, 0))
  dv_spec = pl.BlockSpec((_R, 1), lambda i: (i, 0))
  in_specs = [half_spec, half_spec, dv_spec]
  ins = [agg0, agg1, dv]
  if not first:
    in_specs.append(pl.BlockSpec((_R, _DH), lambda i: (i, 0)))
    ins.append(hid)
  in_specs += [
      pl.BlockSpec((_DH, 3 * _DH), lambda i: (0, 0)),
      pl.BlockSpec((_DH, 3 * _DH), lambda i: (0, 0)),
      pl.BlockSpec((1, 3 * _DH), lambda i: (0, 0)),
      pl.BlockSpec((1, 3 * _DH), lambda i: (0, 0)),
      pl.BlockSpec((1, _DH), lambda i: (0, 0)),
  ]
  ins += [WihT, WhhT, bih, bhh, b]
  if not last:
    in_specs += [pl.BlockSpec((_R, _DH), lambda i: (i, 0)), dv_spec]
    ins += [X1, dvn]
  out_specs = [pl.BlockSpec((_R, _DH), lambda i: (i, 0))]
  out_shape = [jax.ShapeDtypeStruct((_N, _DH), jnp.float32)]
  if not last:
    out_specs += [half_spec, half_spec]
    out_shape += [
        jax.ShapeDtypeStruct((_N, _HALF), jnp.float32),
        jax.ShapeDtypeStruct((_N, _HALF), jnp.float32),
    ]
  res = pl.pallas_call(
      body,
      grid=(_N // _R,),
      in_specs=in_specs,
      out_specs=out_specs,
      out_shape=out_shape,
      compiler_params=_tc_params(),
  )(*ins)
  return res[0] if last else res


def kernel(feat, graphs, W1, b1, W2, b2, Wih1, Whh1, bih1, bhh1,
           Wih2, Whh2, bih2, bhh2):
  rows_all = graphs[:, 0, :].reshape(_T * _E // _CHD, _CHD)
  deg_part = _sc_degree(rows_all)
  dinv = _tc_dinv(deg_part).reshape(_T, _TSTRIDE)

  X1 = _tc_x1(feat, W1)
  Wih1T, Whh1T = Wih1.T, Whh1.T
  Wih2T, Whh2T = Wih2.T, Whh2.T
  bih1r, bhh1r = bih1.reshape(1, -1), bhh1.reshape(1, -1)
  bih2r, bhh2r = bih2.reshape(1, -1), bhh2.reshape(1, -1)
  b1r, b2r = b1.reshape(1, -1), b2.reshape(1, -1)

  y0, y1h = _tc_prep(X1, dinv[0].reshape(_TSTRIDE, 1))
  g1 = None
  g2 = None
  outs = []
  for t in range(_T):
    r3 = graphs[t, 0].reshape(_NSUP, _KB, _CH)
    c3 = graphs[t, 1].reshape(_NSUP, _KB, _CH)
    dv = dinv[t].reshape(_TSTRIDE, 1)

    agg0, agg1 = _sc_scatter(r3, c3, y0, y1h)
    g1, z0, z1 = _tc_fused1(t == 0, agg0, agg1, dv, g1,
                            Wih1T, Whh1T, bih1r, bhh1r, b1r, W2)

    agg20, agg21 = _sc_scatter(r3, c3, z0, z1)
    if t < _T - 1:
      dvn = dinv[t + 1].reshape(_TSTRIDE, 1)
      g2, y0, y1h = _tc_fused2(t == 0, False, agg20, agg21, dv, g2,
                               Wih2T, Whh2T, bih2r, bhh2r, b2r, X1, dvn)
    else:
      g2 = _tc_fused2(t == 0, True, agg20, agg21, dv, g2,
                      Wih2T, Whh2T, bih2r, bhh2r, b2r, None, None)
    outs.append(g2)
  return jnp.stack(outs)


# speedup vs baseline: 15.9900x; 1.0075x over previous
import functools

import jax
import jax.numpy as jnp
from jax import lax
from jax.experimental import pallas as pl
from jax.experimental.pallas import tpu as pltpu
from jax.experimental.pallas import tpu_sc as plsc

_N = 10000
_E = 320000
_T = 3
_DIN = 128
_DH = 256
_HALF = 128

_NC = 2
_NS = 16
_LANES = 16

_TSTRIDE = 10240
_CHD = 2000
_EPT_DEG = _E // (_NC * _NS)

_CH = 80
_KB = 2
_SUP = _CH * _KB
_NSUP = _E // _SUP
_NCPT = _NSUP // _NS
_MBLK = 5
_YDT = jnp.float32
_RPT = 640


def _sc_mesh():
  return plsc.VectorSubcoreMesh(
      core_axis_name="c", subcore_axis_name="s",
      num_cores=_NC, num_subcores=_NS)


def _sc_degree(rows2d):
  @functools.partial(
      pl.kernel,
      out_type=jax.ShapeDtypeStruct((_NC * _NS, _T * _TSTRIDE), jnp.float32),
      mesh=_sc_mesh(),
      scratch_types=[
          pltpu.VMEM((_CHD,), jnp.int32),
          pltpu.VMEM((_T * _TSTRIDE,), jnp.float32),
      ],
      compiler_params=pltpu.CompilerParams(needs_layout_passes=False),
  )
  def k(rows_hbm, out_hbm, idx_v, deg_v):
    cid = lax.axis_index("c")
    sid = lax.axis_index("s")
    wid = sid * _NC + cid
    zeros16 = jnp.zeros((_LANES,), jnp.float32)
    ones16 = jnp.ones((_LANES,), jnp.float32)

    @pl.loop(0, _T * _TSTRIDE // _LANES)
    def _(i):
      deg_v[pl.ds(i * _LANES, _LANES)] = zeros16

    for t in range(_T):
      base_blk = t * (_E // _CHD) + wid * (_EPT_DEG // _CHD)
      off16 = jnp.full((_LANES,), t * _TSTRIDE, jnp.int32)
      for b in range(_EPT_DEG // _CHD):
        pltpu.sync_copy(rows_hbm.at[base_blk + b], idx_v)

        @pl.loop(0, _CHD // _LANES)
        def _(j):
          v = idx_v[pl.ds(j * _LANES, _LANES)] + off16
          plsc.addupdate_scatter(deg_v, [v], ones16)

    pltpu.sync_copy(deg_v, out_hbm.at[wid])

  return k(rows2d)


def _sc_scatter(rows3d, cols3d, y0, y1):
  @functools.partial(
      pl.kernel,
      out_type=[
          jax.ShapeDtypeStruct((_N, _HALF), _YDT),
          jax.ShapeDtypeStruct((_N, _HALF), _YDT),
      ],
      mesh=_sc_mesh(),
      scratch_types=[
          pltpu.VMEM_SHARED((_N, _HALF), _YDT),
          pltpu.VMEM((_MBLK, _KB, _CH), jnp.int32),
          pltpu.VMEM((_MBLK, _KB, _CH), jnp.int32),
          [pltpu.VMEM((_CH, _HALF), _YDT) for _ in range(_KB)],
          [pltpu.VMEM((_CH, _HALF), _YDT) for _ in range(_KB)],
          pltpu.SemaphoreType.DMA,
          pltpu.SemaphoreType.DMA,
          pltpu.SemaphoreType.DMA,
          pltpu.SemaphoreType.DMA,
      ],
      compiler_params=pltpu.CompilerParams(needs_layout_passes=False),
  )
  def k(rows_hbm, cols_hbm, y0_hbm, y1_hbm, agg0_hbm, agg1_hbm,
        acc, idx_r, idx_c, gat_a, gat_b,
        gsem_a, gsem_b, ssem_a, ssem_b):
    cid = lax.axis_index("c")
    sid = lax.axis_index("s")
    last = _NS - 1
    tail = _N - last * _RPT

    def init_acc(y_hbm):
      @pl.when(sid < last)
      def _():
        pltpu.sync_copy(y_hbm.at[pl.ds(sid * _RPT, _RPT)],
                        acc.at[pl.ds(sid * _RPT, _RPT)])
      @pl.when(sid == last)
      def _():
        pltpu.sync_copy(y_hbm.at[pl.ds(last * _RPT, tail)],
                        acc.at[pl.ds(last * _RPT, tail)])

    def edge_loop(y_hbm):
      def fire_gather(m, gat, gsem):
        for j in range(_KB):
          pltpu.async_copy(y_hbm.at[idx_r.at[m, j]], gat[j], gsem)

      def fire_scatter(m, gat, ssem):
        for j in range(_KB):
          pltpu.async_copy(gat[j], acc.at[idx_c.at[m, j]], ssem, add=True)

      def drain(gat, sem):
        for j in range(_KB):
          pltpu.make_async_copy(y_hbm.at[pl.ds(0, _CH)], gat[j], sem).wait()

      @pl.loop(0, _NCPT // _MBLK)
      def _(bi):
        base = sid * _NCPT + bi * _MBLK
        pltpu.sync_copy(rows_hbm.at[pl.ds(base, _MBLK)], idx_r)
        pltpu.sync_copy(cols_hbm.at[pl.ds(base, _MBLK)], idx_c)
        fire_gather(0, gat_a, gsem_a)

        @pl.loop(0, (_MBLK + 1) // 2)
        def _(k):
          drain(gat_a, gsem_a)
          fire_scatter(2 * k, gat_a, ssem_a)
          @pl.when(k > 0)
          def _():
            drain(gat_b, ssem_b)
          @pl.when(2 * k + 1 < _MBLK)
          def _():
            fire_gather(2 * k + 1, gat_b, gsem_b)
          drain(gat_a, ssem_a)
          @pl.when(2 * k + 2 < _MBLK)
          def _():
            fire_gather(2 * k + 2, gat_a, gsem_a)
          @pl.when(2 * k + 1 < _MBLK)
          def _():
            drain(gat_b, gsem_b)
            fire_scatter(2 * k + 1, gat_b, ssem_b)

    def copy_out(agg_hbm):
      @pl.when(sid < last)
      def _():
        pltpu.sync_copy(acc.at[pl.ds(sid * _RPT, _RPT)],
                        agg_hbm.at[pl.ds(sid * _RPT, _RPT)])
      @pl.when(sid == last)
      def _():
        pltpu.sync_copy(acc.at[pl.ds(last * _RPT, tail)],
                        agg_hbm.at[pl.ds(last * _RPT, tail)])

    @pl.when(cid == 0)
    def _():
      init_acc(y0_hbm)
    @pl.when(cid == 1)
    def _():
      init_acc(y1_hbm)
    plsc.subcore_barrier()
    @pl.when(cid == 0)
    def _():
      edge_loop(y0_hbm)
    @pl.when(cid == 1)
    def _():
      edge_loop(y1_hbm)
    plsc.subcore_barrier()
    @pl.when(cid == 0)
    def _():
      copy_out(agg0_hbm)
    @pl.when(cid == 1)
    def _():
      copy_out(agg1_hbm)

  return k(rows3d, cols3d, y0, y1)


_R = 1000


def _tc_params():
  return pltpu.CompilerParams(dimension_semantics=("parallel",))


def _tc_dinv(deg_part):
  blk = 1280
  grid = (_T * _TSTRIDE) // blk

  def body(p_ref, o_ref):
    s = jnp.sum(p_ref[...], axis=0, keepdims=True) + 1.0
    o_ref[...] = lax.rsqrt(s)

  return pl.pallas_call(
      body,
      grid=(grid,),
      in_specs=[pl.BlockSpec((_NC * _NS, blk), lambda i: (0, i))],
      out_specs=pl.BlockSpec((1, blk), lambda i: (0, i)),
      out_shape=jax.ShapeDtypeStruct((1, _T * _TSTRIDE), jnp.float32),
      compiler_params=_tc_params(),
  )(deg_part)


def _tc_x1(feat, W1, dv0):
  def body(f_ref, w_ref, d_ref, o_ref, y0_ref, y1_ref):
    x1 = jnp.dot(f_ref[...], w_ref[...], preferred_element_type=jnp.float32)
    o_ref[...] = x1
    y = (x1 * d_ref[...]).astype(_YDT)
    y0_ref[...] = y[:, :_HALF]
    y1_ref[...] = y[:, _HALF:]

  return pl.pallas_call(
      body,
      grid=(_N // _R,),
      in_specs=[
          pl.BlockSpec((_R, _DIN), lambda i: (i, 0)),
          pl.BlockSpec((_DIN, _DH), lambda i: (0, 0)),
          pl.BlockSpec((_R, 1), lambda i: (i, 0)),
      ],
      out_specs=[
          pl.BlockSpec((_R, _DH), lambda i: (i, 0)),
          pl.BlockSpec((_R, _HALF), lambda i: (i, 0)),
          pl.BlockSpec((_R, _HALF), lambda i: (i, 0)),
      ],
      out_shape=[
          jax.ShapeDtypeStruct((_N, _DH), jnp.float32),
          jax.ShapeDtypeStruct((_N, _HALF), _YDT),
          jax.ShapeDtypeStruct((_N, _HALF), _YDT),
      ],
      compiler_params=_tc_params(),
  )(feat, W1, dv0)


def _gru_math(h, hid, WihT, WhhT, bih, bhh):
  gi = jnp.dot(h.astype(jnp.bfloat16), WihT,
               preferred_element_type=jnp.float32) + bih
  gh = jnp.dot(hid.astype(jnp.bfloat16), WhhT,
               preferred_element_type=jnp.float32) + bhh
  r = jax.nn.sigmoid(gi[:, :_DH] + gh[:, :_DH])
  z = jax.nn.sigmoid(gi[:, _DH:2 * _DH] + gh[:, _DH:2 * _DH])
  n = jnp.tanh(gi[:, 2 * _DH:] + r * gh[:, 2 * _DH:])
  return (1.0 - z) * n + z * hid


def _tc_fused1(first, agg0, agg1, dv, hid, WihT, WhhT, bih, bhh, b, W2):
  def body(*refs):
    if first:
      (a0_ref, a1_ref, d_ref, wi_ref, wh_ref, bi_ref, bh_ref, b_ref, w2_ref,
       g_ref, z0_ref, z1_ref) = refs
    else:
      (a0_ref, a1_ref, d_ref, hid_ref, wi_ref, wh_ref, bi_ref, bh_ref, b_ref,
       w2_ref, g_ref, z0_ref, z1_ref) = refs
    dvb = d_ref[...]
    agg = jnp.concatenate([a0_ref[...], a1_ref[...]], axis=1)
    h = agg.astype(jnp.float32) * dvb + b_ref[...]
    hidv = h if first else hid_ref[...]
    g = _gru_math(h, hidv, wi_ref[...], wh_ref[...], bi_ref[...], bh_ref[...])
    g_ref[...] = g
    y2 = jnp.dot(g.astype(jnp.bfloat16), w2_ref[...],
                 preferred_element_type=jnp.float32) * dvb
    y2 = y2.astype(_YDT)
    z0_ref[...] = y2[:, :_HALF]
    z1_ref[...] = y2[:, _HALF:]

  half_spec = pl.BlockSpec((_R, _HALF), lambda i: (i, 0))
  in_specs = [half_spec, half_spec, pl.BlockSpec((_R, 1), lambda i: (i, 0))]
  ins = [agg0, agg1, dv]
  if not first:
    in_specs.append(pl.BlockSpec((_R, _DH), lambda i: (i, 0)))
    ins.append(hid)
  in_specs += [
      pl.BlockSpec((_DH, 3 * _DH), lambda i: (0, 0)),
      pl.BlockSpec((_DH, 3 * _DH), lambda i: (0, 0)),
      pl.BlockSpec((1, 3 * _DH), lambda i: (0, 0)),
      pl.BlockSpec((1, 3 * _DH), lambda i: (0, 0)),
      pl.BlockSpec((1, _DH), lambda i: (0, 0)),
      pl.BlockSpec((_DH, _DH), lambda i: (0, 0)),
  ]
  ins += [WihT, WhhT, bih, bhh, b, W2]
  return pl.pallas_call(
      body,
      grid=(_N // _R,),
      in_specs=in_specs,
      out_specs=[
          pl.BlockSpec((_R, _DH), lambda i: (i, 0)),
          half_spec,
          half_spec,
      ],
      out_shape=[
          jax.ShapeDtypeStruct((_N, _DH), jnp.float32),
          jax.ShapeDtypeStruct((_N, _HALF), _YDT),
          jax.ShapeDtypeStruct((_N, _HALF), _YDT),
      ],
      compiler_params=_tc_params(),
  )(*ins)


def _tc_fused2(first, last, agg0, agg1, dv, hid, WihT, WhhT, bih, bhh, b,
               X1, dvn):
  def body(*refs):
    refs = list(refs)
    a0_ref = refs.pop(0)
    a1_ref = refs.pop(0)
    d_ref = refs.pop(0)
    hid_ref = None if first else refs.pop(0)
    wi_ref = refs.pop(0)
    wh_ref = refs.pop(0)
    bi_ref = refs.pop(0)
    bh_ref = refs.pop(0)
    b_ref = refs.pop(0)
    if not last:
      x1_ref = refs.pop(0)
      dn_ref = refs.pop(0)
    g_ref = refs.pop(0)
    dvb = d_ref[...]
    agg = jnp.concatenate([a0_ref[...], a1_ref[...]], axis=1)
    h = agg.astype(jnp.float32) * dvb + b_ref[...]
    hidv = h if first else hid_ref[...]
    g = _gru_math(h, hidv, wi_ref[...], wh_ref[...], bi_ref[...], bh_ref[...])
    g_ref[...] = g
    if not last:
      y0_ref = refs.pop(0)
      y1_ref = refs.pop(0)
      yn = (x1_ref[...] * dn_ref[...]).astype(_YDT)
      y0_ref[...] = yn[:, :_HALF]
      y1_ref[...] = yn[:, _HALF:]

  half_spec = pl.BlockSpec((_R, _HALF), lambda i: (i, 0))
  dv_spec = pl.BlockSpec((_R, 1), lambda i: (i, 0))
  in_specs = [half_spec, half_spec, dv_spec]
  ins = [agg0, agg1, dv]
  if not first:
    in_specs.append(pl.BlockSpec((_R, _DH), lambda i: (i, 0)))
    ins.append(hid)
  in_specs += [
      pl.BlockSpec((_DH, 3 * _DH), lambda i: (0, 0)),
      pl.BlockSpec((_DH, 3 * _DH), lambda i: (0, 0)),
      pl.BlockSpec((1, 3 * _DH), lambda i: (0, 0)),
      pl.BlockSpec((1, 3 * _DH), lambda i: (0, 0)),
      pl.BlockSpec((1, _DH), lambda i: (0, 0)),
  ]
  ins += [WihT, WhhT, bih, bhh, b]
  if not last:
    in_specs += [pl.BlockSpec((_R, _DH), lambda i: (i, 0)), dv_spec]
    ins += [X1, dvn]
  out_specs = [pl.BlockSpec((_R, _DH), lambda i: (i, 0))]
  out_shape = [jax.ShapeDtypeStruct((_N, _DH), jnp.float32)]
  if not last:
    out_specs += [half_spec, half_spec]
    out_shape += [
        jax.ShapeDtypeStruct((_N, _HALF), _YDT),
        jax.ShapeDtypeStruct((_N, _HALF), _YDT),
    ]
  res = pl.pallas_call(
      body,
      grid=(_N // _R,),
      in_specs=in_specs,
      out_specs=out_specs,
      out_shape=out_shape,
      compiler_params=_tc_params(),
  )(*ins)
  return res[0] if last else res


def kernel(feat, graphs, W1, b1, W2, b2, Wih1, Whh1, bih1, bhh1,
           Wih2, Whh2, bih2, bhh2):
  rows_all = graphs[:, 0, :].reshape(_T * _E // _CHD, _CHD)
  deg_part = _sc_degree(rows_all)
  dinv = _tc_dinv(deg_part).reshape(_T, _TSTRIDE)

  bf = jnp.bfloat16
  Wih1T, Whh1T = Wih1.T.astype(bf), Whh1.T.astype(bf)
  Wih2T, Whh2T = Wih2.T.astype(bf), Whh2.T.astype(bf)
  W2b = W2.astype(bf)
  bih1r, bhh1r = bih1.reshape(1, -1), bhh1.reshape(1, -1)
  bih2r, bhh2r = bih2.reshape(1, -1), bhh2.reshape(1, -1)
  b1r, b2r = b1.reshape(1, -1), b2.reshape(1, -1)

  X1, y0, y1h = _tc_x1(feat, W1, dinv[0].reshape(_TSTRIDE, 1))
  g1 = None
  g2 = None
  outs = []
  for t in range(_T):
    r3 = graphs[t, 0].reshape(_NSUP, _KB, _CH)
    c3 = graphs[t, 1].reshape(_NSUP, _KB, _CH)
    dv = dinv[t].reshape(_TSTRIDE, 1)

    agg0, agg1 = _sc_scatter(r3, c3, y0, y1h)
    g1, z0, z1 = _tc_fused1(t == 0, agg0, agg1, dv, g1,
                            Wih1T, Whh1T, bih1r, bhh1r, b1r, W2b)

    agg20, agg21 = _sc_scatter(r3, c3, z0, z1)
    if t < _T - 1:
      dvn = dinv[t + 1].reshape(_TSTRIDE, 1)
      g2, y0, y1h = _tc_fused2(t == 0, False, agg20, agg21, dv, g2,
                               Wih2T, Whh2T, bih2r, bhh2r, b2r, X1, dvn)
    else:
      g2 = _tc_fused2(t == 0, True, agg20, agg21, dv, g2,
                      Wih2T, Whh2T, bih2r, bhh2r, b2r, None, None)
    outs.append(g2)
  return jnp.stack(outs)
